# baseline (speedup 1.0000x reference)
import functools

import jax
import jax.numpy as jnp
from jax import lax
from jax.experimental import pallas as pl
from jax.experimental.pallas import tpu as pltpu
from jax.experimental.pallas import tpu_sc as plsc

_NUM_CORES = 2
_NUM_SUBCORES = 16
_LANES = 16


_HB = 8192


def _tc_pack(emb, v_pad):
    V, D = emb.shape
    embT = emb.T

    def body(x_ref, o_ref):
        z = jnp.concatenate([x_ref[:, 0:_HB], x_ref[:, _HB : 2 * _HB]], axis=0)
        o_ref[...] = jnp.transpose(z)

    return pl.pallas_call(
        body,
        out_shape=jax.ShapeDtypeStruct((v_pad // 2, 2 * D), jnp.float32),
        grid=(pl.cdiv(V, 2 * _HB),),
        in_specs=[pl.BlockSpec((D, 2 * _HB), lambda i: (0, i))],
        out_specs=pl.BlockSpec((_HB, 2 * D), lambda i: (i, 0)),
        compiler_params=pltpu.CompilerParams(dimension_semantics=("parallel",)),
    )(embT)


def _sc_gather_minmax(input_bags, emb_rm):
    B, L = input_bags.shape
    V, D = emb_rm.shape
    NW = _NUM_CORES * _NUM_SUBCORES
    assert B % NW == 0
    b_per_w = B // NW
    assert b_per_w % 2 == 0
    nchunk = D // _LANES
    if L > 128:
        l0, l1 = 128, L - 128
    else:
        l0, l1 = L, 0

    mesh = plsc.VectorSubcoreMesh(core_axis_name="c", subcore_axis_name="s")

    @functools.partial(
        pl.kernel,
        out_type=jax.ShapeDtypeStruct((B, 2 * D), jnp.float32),
        mesh=mesh,
        compiler_params=pltpu.CompilerParams(use_tc_tiling_on_sc=False),
        scratch_types=[
            pltpu.VMEM((b_per_w, L), jnp.int32),
            pltpu.VMEM((b_per_w, L), jnp.int32),
            pltpu.VMEM((L, D), jnp.float32),
            pltpu.VMEM((L, D), jnp.float32),
            pltpu.VMEM((b_per_w, 2 * D), jnp.float32),
            pltpu.SemaphoreType.DMA,
            pltpu.SemaphoreType.DMA,
        ],
    )
    def k(bags_hbm, emb_hbm, out_hbm, raw_v, idx_v, rows0, rows1, hid_v, sem0, sem1):
        wid = lax.axis_index("s") * _NUM_CORES + lax.axis_index("c")
        base = wid * b_per_w
        pltpu.sync_copy(bags_hbm.at[pl.ds(base, b_per_w)], raw_v)

        hi_mask = jnp.int32(~(2 * _HB - 1))
        lo_mask = jnp.int32(_HB - 1)

        def remap_chunk(r, c0):
            v = raw_v[r, pl.ds(c0, _LANES)]
            l = (
                (v & hi_mask)
                | ((v & lo_mask) << 1)
                | ((v >> jnp.int32(13)) & jnp.int32(1))
            )
            idx_v[r, pl.ds(c0, _LANES)] = l

        @pl.loop(0, b_per_w)
        def _(r):
            @pl.loop(0, (L // _LANES) * _LANES, step=_LANES)
            def _(c0):
                remap_chunk(r, c0)

            if L % _LANES:
                remap_chunk(r, L - _LANES)

        def start_gather(i, rows, sem):
            pltpu.make_async_copy(
                emb_hbm.at[idx_v.at[i, pl.ds(0, l0)]], rows.at[pl.ds(0, l0)], sem
            ).start()
            if l1:
                pltpu.make_async_copy(
                    emb_hbm.at[idx_v.at[i, pl.ds(l0, l1)]],
                    rows.at[pl.ds(l0, l1)],
                    sem,
                ).start()

        def wait_gather(rows, sem):
            pltpu.make_async_copy(
                emb_hbm.at[idx_v.at[0, pl.ds(0, l0)]], rows.at[pl.ds(0, l0)], sem
            ).wait()
            if l1:
                pltpu.make_async_copy(
                    emb_hbm.at[idx_v.at[0, pl.ds(l0, l1)]],
                    rows.at[pl.ds(l0, l1)],
                    sem,
                ).wait()

        def reduce_bag(rows, i):
            def body(j, carry):
                out_mn = []
                out_mx = []
                for c in range(nchunk):
                    r = rows[j, pl.ds(c * _LANES, _LANES)]
                    out_mn.append(jnp.minimum(carry[c], r))
                    out_mx.append(jnp.maximum(carry[nchunk + c], r))
                return tuple(out_mn) + tuple(out_mx)

            init = tuple(rows[0, pl.ds(c * _LANES, _LANES)] for c in range(nchunk))
            carry = lax.fori_loop(1, L, body, init + init, unroll=8)
            for c in range(nchunk):
                hid_v[i, pl.ds(c * _LANES, _LANES)] = carry[c]
                hid_v[i, pl.ds(D + c * _LANES, _LANES)] = carry[nchunk + c]

        start_gather(0, rows0, sem0)

        @pl.loop(0, b_per_w, step=2)
        def _(i):
            wait_gather(rows0, sem0)
            start_gather(i + 1, rows1, sem1)
            reduce_bag(rows0, i)
            wait_gather(rows1, sem1)

            @pl.when(i + 2 < b_per_w)
            def _():
                start_gather(i + 2, rows0, sem0)

            reduce_bag(rows1, i + 1)

        pltpu.sync_copy(hid_v, out_hbm.at[pl.ds(base, b_per_w)])

    return k(input_bags, emb_rm)


def _tc_head(hidden, W, b):
    B, H = hidden.shape
    C = W.shape[0]

    def body(h_ref, w_ref, b_ref, o_ref):
        h = h_ref[...]
        w = w_ref[...]
        logits = lax.dot_general(
            h, w, (((1,), (1,)), ((), ())), preferred_element_type=jnp.float32
        )
        logits = logits + b_ref[...]
        m = jnp.max(logits, axis=1, keepdims=True)
        x = logits - m
        lse = jnp.log(jnp.sum(jnp.exp(x), axis=1, keepdims=True))
        o_ref[...] = x - lse

    return pl.pallas_call(
        body,
        out_shape=jax.ShapeDtypeStruct((B, C), jnp.float32),
    )(hidden, W, b.reshape(1, C))


def kernel(input_bags, emb, W, b):
    V, D = emb.shape
    v_pad = 1 << 20
    packed = _tc_pack(emb, v_pad)
    emb_rm = jnp.reshape(packed, (v_pad, D))
    hidden = _sc_gather_minmax(input_bags.astype(jnp.int32), emb_rm)
    return _tc_head(hidden, W, b)


# speedup vs baseline: 1.0185x; 1.0185x over previous
import functools

import jax
import jax.numpy as jnp
from jax import lax
from jax.experimental import pallas as pl
from jax.experimental.pallas import tpu as pltpu
from jax.experimental.pallas import tpu_sc as plsc

_NUM_CORES = 2
_NUM_SUBCORES = 16
_LANES = 16


_HB = 16384
_HB_LOG = _HB.bit_length() - 1


def _tc_pack(emb, v_pad):
    V, D = emb.shape
    embT = emb.T

    def body(x_ref, o_ref):
        z = jnp.concatenate([x_ref[:, 0:_HB], x_ref[:, _HB : 2 * _HB]], axis=0)
        o_ref[...] = jnp.transpose(z)

    return pl.pallas_call(
        body,
        out_shape=jax.ShapeDtypeStruct((v_pad // 2, 2 * D), jnp.float32),
        grid=(pl.cdiv(V, 2 * _HB),),
        in_specs=[pl.BlockSpec((D, 2 * _HB), lambda i: (0, i))],
        out_specs=pl.BlockSpec((_HB, 2 * D), lambda i: (i, 0)),
        compiler_params=pltpu.CompilerParams(dimension_semantics=("parallel",)),
    )(embT)


def _sc_gather_minmax(input_bags, emb_rm):
    B, L = input_bags.shape
    V, D = emb_rm.shape
    NW = _NUM_CORES * _NUM_SUBCORES
    assert B % NW == 0
    b_per_w = B // NW
    assert b_per_w % 2 == 0
    nchunk = D // _LANES
    if L > 128:
        l0, l1 = 128, L - 128
    else:
        l0, l1 = L, 0

    mesh = plsc.VectorSubcoreMesh(core_axis_name="c", subcore_axis_name="s")

    @functools.partial(
        pl.kernel,
        out_type=jax.ShapeDtypeStruct((B, 2 * D), jnp.float32),
        mesh=mesh,
        compiler_params=pltpu.CompilerParams(use_tc_tiling_on_sc=False),
        scratch_types=[
            pltpu.VMEM((b_per_w, L), jnp.int32),
            pltpu.VMEM((b_per_w, L), jnp.int32),
            pltpu.VMEM((L, D), jnp.float32),
            pltpu.VMEM((L, D), jnp.float32),
            pltpu.VMEM((b_per_w, 2 * D), jnp.float32),
            pltpu.SemaphoreType.DMA,
            pltpu.SemaphoreType.DMA,
        ],
    )
    def k(bags_hbm, emb_hbm, out_hbm, raw_v, idx_v, rows0, rows1, hid_v, sem0, sem1):
        wid = lax.axis_index("s") * _NUM_CORES + lax.axis_index("c")
        base = wid * b_per_w
        pltpu.sync_copy(bags_hbm.at[pl.ds(base, b_per_w)], raw_v)

        hi_mask = jnp.int32(~(2 * _HB - 1))
        lo_mask = jnp.int32(_HB - 1)

        def remap_chunk(r, c0):
            v = raw_v[r, pl.ds(c0, _LANES)]
            l = (
                (v & hi_mask)
                | ((v & lo_mask) << 1)
                | ((v >> jnp.int32(_HB_LOG)) & jnp.int32(1))
            )
            idx_v[r, pl.ds(c0, _LANES)] = l

        @pl.loop(0, b_per_w)
        def _(r):
            @pl.loop(0, (L // _LANES) * _LANES, step=_LANES)
            def _(c0):
                remap_chunk(r, c0)

            if L % _LANES:
                remap_chunk(r, L - _LANES)

        def start_gather(i, rows, sem):
            pltpu.make_async_copy(
                emb_hbm.at[idx_v.at[i, pl.ds(0, l0)]], rows.at[pl.ds(0, l0)], sem
            ).start()
            if l1:
                pltpu.make_async_copy(
                    emb_hbm.at[idx_v.at[i, pl.ds(l0, l1)]],
                    rows.at[pl.ds(l0, l1)],
                    sem,
                ).start()

        def wait_gather(rows, sem):
            pltpu.make_async_copy(
                emb_hbm.at[idx_v.at[0, pl.ds(0, l0)]], rows.at[pl.ds(0, l0)], sem
            ).wait()
            if l1:
                pltpu.make_async_copy(
                    emb_hbm.at[idx_v.at[0, pl.ds(l0, l1)]],
                    rows.at[pl.ds(l0, l1)],
                    sem,
                ).wait()

        def reduce_bag(rows, i):
            def body(j, carry):
                out_mn = []
                out_mx = []
                for c in range(nchunk):
                    r = rows[j, pl.ds(c * _LANES, _LANES)]
                    out_mn.append(jnp.minimum(carry[c], r))
                    out_mx.append(jnp.maximum(carry[nchunk + c], r))
                return tuple(out_mn) + tuple(out_mx)

            init = tuple(rows[0, pl.ds(c * _LANES, _LANES)] for c in range(nchunk))
            carry = lax.fori_loop(1, L, body, init + init, unroll=8)
            for c in range(nchunk):
                hid_v[i, pl.ds(c * _LANES, _LANES)] = carry[c]
                hid_v[i, pl.ds(D + c * _LANES, _LANES)] = carry[nchunk + c]

        start_gather(0, rows0, sem0)

        @pl.loop(0, b_per_w, step=2)
        def _(i):
            wait_gather(rows0, sem0)
            start_gather(i + 1, rows1, sem1)
            reduce_bag(rows0, i)
            wait_gather(rows1, sem1)

            @pl.when(i + 2 < b_per_w)
            def _():
                start_gather(i + 2, rows0, sem0)

            reduce_bag(rows1, i + 1)

        pltpu.sync_copy(hid_v, out_hbm.at[pl.ds(base, b_per_w)])

    return k(input_bags, emb_rm)


def _tc_head(hidden, W, b):
    B, H = hidden.shape
    C = W.shape[0]

    def body(h_ref, w_ref, b_ref, o_ref):
        h = h_ref[...]
        w = w_ref[...]
        logits = lax.dot_general(
            h, w, (((1,), (1,)), ((), ())), preferred_element_type=jnp.float32
        )
        logits = logits + b_ref[...]
        m = jnp.max(logits, axis=1, keepdims=True)
        x = logits - m
        lse = jnp.log(jnp.sum(jnp.exp(x), axis=1, keepdims=True))
        o_ref[...] = x - lse

    return pl.pallas_call(
        body,
        out_shape=jax.ShapeDtypeStruct((B, C), jnp.float32),
    )(hidden, W, b.reshape(1, C))


def kernel(input_bags, emb, W, b):
    V, D = emb.shape
    v_pad = 1 << 20
    packed = _tc_pack(emb, v_pad)
    emb_rm = jnp.reshape(packed, (v_pad, D))
    hidden = _sc_gather_minmax(input_bags.astype(jnp.int32), emb_rm)
    return _tc_head(hidden, W, b)


# speedup vs baseline: 1.3560x; 1.3314x over previous
import functools

import jax
import jax.numpy as jnp
from jax import lax
from jax.experimental import pallas as pl
from jax.experimental.pallas import tpu as pltpu
from jax.experimental.pallas import tpu_sc as plsc

_NUM_CORES = 2
_NUM_SUBCORES = 16
_LANES = 16


_HB = 16384
_HB_LOG = _HB.bit_length() - 1


def _tc_pack(emb, v_pad):
    V, D = emb.shape
    embT = emb.T

    def body(x_ref, o_ref):
        z = jnp.concatenate([x_ref[:, 0:_HB], x_ref[:, _HB : 2 * _HB]], axis=0)
        o_ref[...] = jnp.transpose(z)

    return pl.pallas_call(
        body,
        out_shape=jax.ShapeDtypeStruct((v_pad // 2, 2 * D), jnp.float32),
        grid=(pl.cdiv(V, 2 * _HB),),
        in_specs=[pl.BlockSpec((D, 2 * _HB), lambda i: (0, i))],
        out_specs=pl.BlockSpec((_HB, 2 * D), lambda i: (i, 0)),
        compiler_params=pltpu.CompilerParams(dimension_semantics=("parallel",)),
    )(embT)


def _sc_gather_minmax(input_bags, emb_rm):
    B, L = input_bags.shape
    V, D = emb_rm.shape
    NW = _NUM_CORES * _NUM_SUBCORES
    assert B % NW == 0
    b_per_w = B // NW
    assert b_per_w % 2 == 0
    nchunk = D // _LANES
    if L > 128:
        l0, l1 = 128, L - 128
    else:
        l0, l1 = L, 0

    mesh = plsc.VectorSubcoreMesh(core_axis_name="c", subcore_axis_name="s")

    @functools.partial(
        pl.kernel,
        out_type=jax.ShapeDtypeStruct((B, 2 * D), jnp.float32),
        mesh=mesh,
        compiler_params=pltpu.CompilerParams(use_tc_tiling_on_sc=False),
        scratch_types=[
            pltpu.VMEM((b_per_w, ((L + _LANES - 1) // _LANES) * _LANES), jnp.int32),
            pltpu.VMEM((L, D), jnp.float32),
            pltpu.VMEM((L, D), jnp.float32),
            pltpu.VMEM((L, D), jnp.float32),
            pltpu.VMEM((L, D), jnp.float32),
            pltpu.VMEM((b_per_w, 2 * D), jnp.float32),
            pltpu.SemaphoreType.DMA,
            pltpu.SemaphoreType.DMA,
            pltpu.SemaphoreType.DMA,
            pltpu.SemaphoreType.DMA,
        ],
    )
    def k(bags_hbm, emb_hbm, out_hbm, idx_v, r0, r1, r2, r3, hid_v, s0, s1, s2, s3):
        wid = lax.axis_index("s") * _NUM_CORES + lax.axis_index("c")
        base = wid * b_per_w
        lp = idx_v.shape[1]
        pltpu.sync_copy(bags_hbm.at[pl.ds(base, b_per_w)], idx_v.at[:, pl.ds(0, L)])

        hi_mask = jnp.int32(~(2 * _HB - 1))
        lo_mask = jnp.int32(_HB - 1)

        @pl.loop(0, b_per_w)
        def _(r):
            @pl.loop(0, lp, step=_LANES)
            def _(c0):
                v = idx_v[r, pl.ds(c0, _LANES)]
                l = (
                    (v & hi_mask)
                    | ((v & lo_mask) << 1)
                    | ((v >> jnp.int32(_HB_LOG)) & jnp.int32(1))
                )
                idx_v[r, pl.ds(c0, _LANES)] = l

        def start_gather(i, rows, sem):
            pltpu.make_async_copy(
                emb_hbm.at[idx_v.at[i, pl.ds(0, l0)]], rows.at[pl.ds(0, l0)], sem
            ).start()
            if l1:
                pltpu.make_async_copy(
                    emb_hbm.at[idx_v.at[i, pl.ds(l0, l1)]],
                    rows.at[pl.ds(l0, l1)],
                    sem,
                ).start()

        def wait_gather(rows, sem):
            pltpu.make_async_copy(
                emb_hbm.at[idx_v.at[0, pl.ds(0, l0)]], rows.at[pl.ds(0, l0)], sem
            ).wait()
            if l1:
                pltpu.make_async_copy(
                    emb_hbm.at[idx_v.at[0, pl.ds(l0, l1)]],
                    rows.at[pl.ds(l0, l1)],
                    sem,
                ).wait()

        def reduce_bag(rows, i):
            def body(j, carry):
                out_mn = []
                out_mx = []
                for c in range(nchunk):
                    r = rows[j, pl.ds(c * _LANES, _LANES)]
                    out_mn.append(jnp.minimum(carry[c], r))
                    out_mx.append(jnp.maximum(carry[nchunk + c], r))
                return tuple(out_mn) + tuple(out_mx)

            init = tuple(rows[0, pl.ds(c * _LANES, _LANES)] for c in range(nchunk))
            carry = lax.fori_loop(1, L, body, init + init, unroll=8)
            for c in range(nchunk):
                hid_v[i, pl.ds(c * _LANES, _LANES)] = carry[c]
                hid_v[i, pl.ds(D + c * _LANES, _LANES)] = carry[nchunk + c]

        bufs = (r0, r1, r2, r3)
        sems = (s0, s1, s2, s3)
        for kk in range(3):
            start_gather(kk, bufs[kk], sems[kk])

        @pl.loop(0, b_per_w, step=4)
        def _(i):
            for kk in range(4):
                wait_gather(bufs[kk], sems[kk])
                nxt = i + kk + 3

                @pl.when(nxt < b_per_w)
                def _(kk=kk, nxt=nxt):
                    start_gather(nxt, bufs[(kk + 3) % 4], sems[(kk + 3) % 4])

                reduce_bag(bufs[kk], i + kk)

        pltpu.sync_copy(hid_v, out_hbm.at[pl.ds(base, b_per_w)])

    return k(input_bags, emb_rm)


def _tc_head(hidden, W, b):
    B, H = hidden.shape
    C = W.shape[0]

    def body(h_ref, w_ref, b_ref, o_ref):
        h = h_ref[...]
        w = w_ref[...]
        logits = lax.dot_general(
            h, w, (((1,), (1,)), ((), ())), preferred_element_type=jnp.float32
        )
        logits = logits + b_ref[...]
        m = jnp.max(logits, axis=1, keepdims=True)
        x = logits - m
        lse = jnp.log(jnp.sum(jnp.exp(x), axis=1, keepdims=True))
        o_ref[...] = x - lse

    return pl.pallas_call(
        body,
        out_shape=jax.ShapeDtypeStruct((B, C), jnp.float32),
    )(hidden, W, b.reshape(1, C))


def kernel(input_bags, emb, W, b):
    V, D = emb.shape
    v_pad = 1 << 20
    packed = _tc_pack(emb, v_pad)
    emb_rm = jnp.reshape(packed, (v_pad, D))
    hidden = _sc_gather_minmax(input_bags.astype(jnp.int32), emb_rm)
    return _tc_head(hidden, W, b)
